# baseline (speedup 1.0000x reference)
import jax
import jax.numpy as jnp
from jax import lax
from jax.experimental import pallas as pl
from jax.experimental.pallas import tpu as pltpu
from jax.experimental.pallas import tpu_sc as plsc

N = 10000
D = 128
E = 320000
NC = 2
NS = 16
L = 16
NW = NC * NS
CHUNK = 96
CPT = 108
STAGE = 18
N_STAGES = CPT // STAGE
EPAD = CPT * NW * CHUNK
NPAD = 10112
ROWS_PER_TILE = NPAD // NS
GROUPS = CHUNK // L


def _prep_body(x_ref, w_ref, b_ref, a_ref, ab_ref, h_ref, s_ref):
    h = jnp.dot(x_ref[...], w_ref[...], preferred_element_type=jnp.float32)
    h = jnp.maximum(h + b_ref[...], 0.0)
    h_ref[...] = h
    s_ref[...] = jnp.dot(h, a_ref[...], preferred_element_type=jnp.float32) + ab_ref[...]


def _combine_body(h_ref, ps_ref, eps_ref, a_ref, ab_ref, hn_ref, s_ref):
    eps = eps_ref[0, 0]
    tot = ps_ref[0, :N] + ps_ref[1, :N]
    h = jnp.maximum(eps * h_ref[...] + (1.0 - eps) * tot, 0.0)
    hn_ref[...] = h
    s_ref[...] = jnp.dot(h, a_ref[...], preferred_element_type=jnp.float32) + ab_ref[...]


def _final_body(h_ref, ps_ref, eps_ref, w_ref, b_ref, o_ref):
    eps = eps_ref[0, 0]
    tot = ps_ref[0, :N] + ps_ref[1, :N]
    h = jnp.maximum(eps * h_ref[...] + (1.0 - eps) * tot, 0.0)
    o_ref[...] = jnp.dot(h, w_ref[...], preferred_element_type=jnp.float32) + b_ref[...]


def _edge_body(h_hbm, s_hbm, row_hbm, col_hbm, out_hbm,
               accum, s_v, row_st, col_st, colb_v, alpha_v, msg_v, sg0, sg1):
    c = lax.axis_index("c")
    sid = lax.axis_index("s")
    t = c * NS + sid
    gsem = (sg0, sg1)

    pltpu.sync_copy(s_hbm, s_v)

    zeros16 = jnp.zeros((L,), jnp.float32)

    def _zero_row(i, carry):
        for d in range(D // L):
            msg_v[0, i, pl.ds(d * L, L)] = zeros16
        return carry

    lax.fori_loop(0, CHUNK, _zero_row, 0)
    for k in range(6):
        pltpu.sync_copy(msg_v.at[0],
                        accum.at[pl.ds(sid * ROWS_PER_TILE + k * CHUNK, CHUNK)])
    pltpu.sync_copy(msg_v.at[0].at[pl.ds(0, 56)],
                    accum.at[pl.ds(sid * ROWS_PER_TILE + 576, 56)])
    plsc.subcore_barrier()

    def _fire_gather(j, b):
        pltpu.async_copy(h_hbm.at[row_st.at[pl.ds(j * CHUNK, CHUNK)]],
                         msg_v.at[b], gsem[b])

    def _wait_gather(j, b):
        pltpu.make_async_copy(h_hbm.at[row_st.at[pl.ds(j * CHUNK, CHUNK)]],
                              msg_v.at[b], gsem[b]).wait()

    def _compute(st, j, b):
        ebase = (t * CPT + st * STAGE + j) * CHUNK
        for i in range(GROUPS):
            ridx = row_st[pl.ds(j * CHUNK + i * L, L)]
            cidx = col_st[pl.ds(j * CHUNK + i * L, L)]
            colb_v[b, pl.ds(i * L, L)] = cidx
            sv1 = plsc.load_gather(s_v, [ridx * 2])
            sv2 = plsc.load_gather(s_v, [cidx * 2 + 1])
            z = sv1 + sv2
            ex = jnp.exp(-2.0 * jnp.abs(z))
            th = (1.0 - ex) / (1.0 + ex)
            th = jnp.where(z < 0.0, -th, th)
            alpha_v[pl.ds(i * L, L)] = jnp.where(ebase + i * L < E, th, 0.0)

        @plsc.parallel_loop(0, CHUNK, unroll=4)
        def _scale(e_i):
            bc = plsc.load_gather(alpha_v, [jnp.full((L,), e_i, jnp.int32)])
            for d in range(D // L):
                msg_v[b, e_i, pl.ds(d * L, L)] = msg_v[b, e_i, pl.ds(d * L, L)] * bc

    def _stage(st, carry):
        base = (t * CPT + st * STAGE) * CHUNK
        pltpu.sync_copy(row_hbm.at[pl.ds(base, STAGE * CHUNK)], row_st)
        pltpu.sync_copy(col_hbm.at[pl.ds(base, STAGE * CHUNK)], col_st)

        def _round(r, carry1):
            for b in range(2):
                j = 2 * r + b
                pltpu.async_copy(h_hbm.at[row_st.at[pl.ds(j * CHUNK, CHUNK)]],
                                 msg_v.at[b], gsem[b]).wait()
                _compute(st, j, b)
                pltpu.sync_copy(msg_v.at[b], accum.at[colb_v.at[b]], add=True)
            return carry1

        lax.fori_loop(0, STAGE // 2, _round, 0)
        return carry

    lax.fori_loop(0, N_STAGES, _stage, 0)
    plsc.subcore_barrier()

    for off, n in ((0, 128), (128, 128), (256, 128), (384, 128), (512, 120)):
        pltpu.sync_copy(accum.at[pl.ds(sid * ROWS_PER_TILE + off, n)],
                        out_hbm.at[c, pl.ds(sid * ROWS_PER_TILE + off, n)])


_edge_call = pl.kernel(
    _edge_body,
    out_type=jax.ShapeDtypeStruct((NC, NPAD, D), jnp.float32),
    mesh=plsc.VectorSubcoreMesh(core_axis_name="c", subcore_axis_name="s"),
    compiler_params=pltpu.CompilerParams(needs_layout_passes=False),
    scratch_types=[
        pltpu.VMEM_SHARED((NPAD, D), jnp.float32),
        pltpu.VMEM((N * 2,), jnp.float32),
        pltpu.VMEM((STAGE * CHUNK,), jnp.int32),
        pltpu.VMEM((STAGE * CHUNK,), jnp.int32),
        pltpu.VMEM((2, CHUNK), jnp.int32),
        pltpu.VMEM((CHUNK,), jnp.float32),
        pltpu.VMEM((2, CHUNK, D), jnp.float32),
        pltpu.SemaphoreType.DMA,
        pltpu.SemaphoreType.DMA,
    ],
)


_prep_call = pl.pallas_call(
    _prep_body,
    out_shape=(jax.ShapeDtypeStruct((N, D), jnp.float32),
               jax.ShapeDtypeStruct((N, 2), jnp.float32)),
)

_combine_call = pl.pallas_call(
    _combine_body,
    out_shape=(jax.ShapeDtypeStruct((N, D), jnp.float32),
               jax.ShapeDtypeStruct((N, 2), jnp.float32)),
)


def _final_call(h, ps, eps, w, b):
    return pl.pallas_call(
        _final_body,
        out_shape=jax.ShapeDtypeStruct((N, w.shape[1]), jnp.float32),
    )(h, ps, eps, w, b)


@jax.jit
def kernel(x, edge_index, W0, b0, attW1, attb1, eps1, attW2, attb2, eps2, Wc, bc):
    row = edge_index[0]
    col = edge_index[1]
    pad = EPAD - E
    zpad = jnp.zeros((pad,), row.dtype)
    rowp = jnp.concatenate([row, zpad])
    colp = jnp.concatenate([col, zpad])

    A1 = attW1.reshape(2, D).T
    bA1 = jnp.concatenate([attb1, jnp.zeros((1,), jnp.float32)]).reshape(1, 2)
    A2 = attW2.reshape(2, D).T
    bA2 = jnp.concatenate([attb2, jnp.zeros((1,), jnp.float32)]).reshape(1, 2)

    h0, s12a = _prep_call(x, W0.T, b0.reshape(1, D), A1, bA1)
    ps1 = _edge_call(h0, s12a.reshape(-1), rowp, colp)
    h1, s12b = _combine_call(h0, ps1, eps1.reshape(1, 1), A2, bA2)
    ps2 = _edge_call(h1, s12b.reshape(-1), rowp, colp)
    return _final_call(h1, ps2, eps2.reshape(1, 1), Wc.T, bc.reshape(1, -1))


# speedup vs baseline: 2.7287x; 2.7287x over previous
import jax
import jax.numpy as jnp
from jax import lax
from jax.experimental import pallas as pl
from jax.experimental.pallas import tpu as pltpu
from jax.experimental.pallas import tpu_sc as plsc

N = 10000
D = 128
E = 320000
NC = 2
NS = 16
L = 16
NW = NC * NS
CHUNK = 128
CHUNKS_TOTAL = E // CHUNK
CHUNKS_PER_TILE = 80
EPAD = CHUNKS_PER_TILE * NW * CHUNK
NPAD = 10112
ROWS_PER_TILE = NPAD // NS
STAGE = 16
N_STAGES = CHUNKS_PER_TILE // STAGE


def _prep_body(x_ref, w_ref, b_ref, a_ref, ab_ref, h_ref, s_ref):
    h = jnp.dot(x_ref[...], w_ref[...], preferred_element_type=jnp.float32)
    h = jnp.maximum(h + b_ref[...], 0.0)
    h_ref[...] = h
    s_ref[...] = jnp.dot(h, a_ref[...], preferred_element_type=jnp.float32) + ab_ref[...]


def _combine_body(h_ref, ps_ref, eps_ref, a_ref, ab_ref, hn_ref, s_ref):
    eps = eps_ref[0, 0]
    tot = ps_ref[0, :N] + ps_ref[1, :N]
    h = jnp.maximum(eps * h_ref[...] + (1.0 - eps) * tot, 0.0)
    hn_ref[...] = h
    s_ref[...] = jnp.dot(h, a_ref[...], preferred_element_type=jnp.float32) + ab_ref[...]


def _final_body(h_ref, ps_ref, eps_ref, w_ref, b_ref, o_ref):
    eps = eps_ref[0, 0]
    tot = ps_ref[0, :N] + ps_ref[1, :N]
    h = jnp.maximum(eps * h_ref[...] + (1.0 - eps) * tot, 0.0)
    o_ref[...] = jnp.dot(h, w_ref[...], preferred_element_type=jnp.float32) + b_ref[...]


def _edge_body(h_hbm, s_hbm, row_hbm, col_hbm, out_hbm,
               accum, s_v, row_v, col_v, alpha_v, msg_v, sem):
    c = lax.axis_index("c")
    sid = lax.axis_index("s")
    t = c * NS + sid

    pltpu.sync_copy(s_hbm, s_v)

    zeros16 = jnp.zeros((L,), jnp.float32)

    def _zero_row(i, carry):
        for d in range(D // L):
            msg_v[i, pl.ds(d * L, L)] = zeros16
        return carry

    lax.fori_loop(0, CHUNK, _zero_row, 0)
    for off, n in ((0, 128), (128, 128), (256, 128), (384, 128), (512, 120)):
        pltpu.sync_copy(msg_v.at[pl.ds(0, n)],
                        accum.at[pl.ds(sid * ROWS_PER_TILE + off, n)])
    plsc.subcore_barrier()

    def _stage(st, carry):
        base = t * CHUNKS_PER_TILE + st * STAGE
        pltpu.sync_copy(row_hbm.at[pl.ds(base, STAGE)], row_v)
        pltpu.sync_copy(col_hbm.at[pl.ds(base, STAGE)], col_v)

        def _chunk(j, carry1):
            g = base + j

            @pl.when(g < CHUNKS_TOTAL)
            def _():
                pltpu.async_copy(h_hbm.at[row_v.at[j]], msg_v, sem).wait()
                @plsc.parallel_loop(0, CHUNK // L, unroll=4)
                def _alpha(i):
                    ridx = row_v[j, pl.ds(i * L, L)]
                    cidx = col_v[j, pl.ds(i * L, L)]
                    sv1 = plsc.load_gather(s_v, [ridx * 2])
                    sv2 = plsc.load_gather(s_v, [cidx * 2 + 1])
                    z = sv1 + sv2
                    ex = jnp.exp(-2.0 * jnp.abs(z))
                    th = (1.0 - ex) / (1.0 + ex)
                    alpha_v[pl.ds(i * L, L)] = jnp.where(z < 0.0, -th, th)

                @plsc.parallel_loop(0, CHUNK, unroll=8)
                def _scale(e_i):
                    bc = plsc.load_gather(alpha_v, [jnp.full((L,), e_i, jnp.int32)])
                    for d in range(D // L):
                        msg_v[e_i, pl.ds(d * L, L)] = msg_v[e_i, pl.ds(d * L, L)] * bc

                pltpu.sync_copy(msg_v, accum.at[col_v.at[j]], add=True)

            return carry1

        lax.fori_loop(0, STAGE, _chunk, 0)
        return carry

    lax.fori_loop(0, N_STAGES, _stage, 0)
    plsc.subcore_barrier()

    for off, n in ((0, 128), (128, 128), (256, 128), (384, 128), (512, 120)):
        pltpu.sync_copy(accum.at[pl.ds(sid * ROWS_PER_TILE + off, n)],
                        out_hbm.at[c, pl.ds(sid * ROWS_PER_TILE + off, n)])


_edge_call = pl.kernel(
    _edge_body,
    out_type=jax.ShapeDtypeStruct((NC, NPAD, D), jnp.float32),
    mesh=plsc.VectorSubcoreMesh(core_axis_name="c", subcore_axis_name="s"),
    compiler_params=pltpu.CompilerParams(needs_layout_passes=False),
    scratch_types=[
        pltpu.VMEM_SHARED((NPAD, D), jnp.float32),
        pltpu.VMEM((N * 2,), jnp.float32),
        pltpu.VMEM((STAGE, CHUNK), jnp.int32),
        pltpu.VMEM((STAGE, CHUNK), jnp.int32),
        pltpu.VMEM((CHUNK,), jnp.float32),
        pltpu.VMEM((CHUNK, D), jnp.float32),
        pltpu.SemaphoreType.DMA,
    ],
)


_prep_call = pl.pallas_call(
    _prep_body,
    out_shape=(jax.ShapeDtypeStruct((N, D), jnp.float32),
               jax.ShapeDtypeStruct((N, 2), jnp.float32)),
)

_combine_call = pl.pallas_call(
    _combine_body,
    out_shape=(jax.ShapeDtypeStruct((N, D), jnp.float32),
               jax.ShapeDtypeStruct((N, 2), jnp.float32)),
)


def _final_call(h, ps, eps, w, b):
    return pl.pallas_call(
        _final_body,
        out_shape=jax.ShapeDtypeStruct((N, w.shape[1]), jnp.float32),
    )(h, ps, eps, w, b)


@jax.jit
def kernel(x, edge_index, W0, b0, attW1, attb1, eps1, attW2, attb2, eps2, Wc, bc):
    row = edge_index[0]
    col = edge_index[1]
    pad = EPAD - E
    rowp = jnp.concatenate([row, jnp.zeros((pad,), row.dtype)]).reshape(-1, CHUNK)
    colp = jnp.concatenate([col, jnp.zeros((pad,), col.dtype)]).reshape(-1, CHUNK)

    A1 = attW1.reshape(2, D).T
    bA1 = jnp.concatenate([attb1, jnp.zeros((1,), jnp.float32)]).reshape(1, 2)
    A2 = attW2.reshape(2, D).T
    bA2 = jnp.concatenate([attb2, jnp.zeros((1,), jnp.float32)]).reshape(1, 2)

    h0, s12a = _prep_call(x, W0.T, b0.reshape(1, D), A1, bA1)
    ps1 = _edge_call(h0, s12a.reshape(-1), rowp, colp)
    h1, s12b = _combine_call(h0, ps1, eps1.reshape(1, 1), A2, bA2)
    ps2 = _edge_call(h1, s12b.reshape(-1), rowp, colp)
    return _final_call(h1, ps2, eps2.reshape(1, 1), Wc.T, bc.reshape(1, -1))
